# baseline (speedup 1.0000x reference)
import functools

import jax
import jax.numpy as jnp
from jax import lax
from jax.experimental import pallas as pl
from jax.experimental.pallas import tpu as pltpu
from jax.experimental.pallas import tpu_sc as plsc

DIM = 16
BATCH = 16384
NUM_CORES = 2
NUM_SUBCORES = 16
NUM_WORKERS = NUM_CORES * NUM_SUBCORES
ROWS_PER_WORKER = BATCH // NUM_WORKERS
CHUNK = 128
CHUNKS_PER_WORKER = ROWS_PER_WORKER // CHUNK
NCHUNKS = BATCH // CHUNK


def _mfbias_body(ui_hbm, ii_hbm, ut_hbm, it_hbm, ub_hbm, ib_hbm, gb_hbm,
                 out_hbm,
                 uidx_v, iidx_v, ue_v, ie_v, ub_v, ib_v, gb_v, out_v, sem):
    wid = lax.axis_index("s") * NUM_CORES + lax.axis_index("c")
    crow0 = wid * CHUNKS_PER_WORKER

    pltpu.sync_copy(ui_hbm.at[pl.ds(crow0, CHUNKS_PER_WORKER)], uidx_v)
    pltpu.sync_copy(ii_hbm.at[pl.ds(crow0, CHUNKS_PER_WORKER)], iidx_v)
    pltpu.sync_copy(gb_hbm, gb_v)

    bias_handles = []
    for j in range(CHUNKS_PER_WORKER):
        dst = pl.ds(j * CHUNK, CHUNK)
        bias_handles.append(pltpu.async_copy(
            ub_hbm.at[uidx_v.at[j]], ub_v.at[dst], sem))
        bias_handles.append(pltpu.async_copy(
            ib_hbm.at[iidx_v.at[j]], ib_v.at[dst], sem))

    gb = gb_v[...]

    for j in range(CHUNKS_PER_WORKER):
        handles = []
        for d in range(DIM):
            handles.append(pltpu.async_copy(
                ut_hbm.at[d].at[uidx_v.at[j]], ue_v.at[d], sem))
            handles.append(pltpu.async_copy(
                it_hbm.at[d].at[iidx_v.at[j]], ie_v.at[d], sem))
        for h in handles:
            h.wait()
        for g in range(CHUNK // 16):
            s = pl.ds(g * 16, 16)
            acc = gb
            for d in range(DIM):
                acc = acc + ue_v[d, s] * ie_v[d, s]
            out_v[pl.ds(j * CHUNK + g * 16, 16)] = acc

    for h in bias_handles:
        h.wait()
    for t in range(ROWS_PER_WORKER // 16):
        s = pl.ds(t * 16, 16)
        out_v[s] = out_v[s] + ub_v[s] + ib_v[s]
    pltpu.sync_copy(out_v, out_hbm.at[pl.ds(wid * ROWS_PER_WORKER,
                                            ROWS_PER_WORKER)])


@functools.partial(jax.jit)
def _mfbias_call(ui2, ii2, ut_t, it_t, user_bias, item_bias, gb16):
    mesh = plsc.VectorSubcoreMesh(core_axis_name="c", subcore_axis_name="s")
    run = pl.kernel(
        _mfbias_body,
        out_type=jax.ShapeDtypeStruct((BATCH,), jnp.float32),
        mesh=mesh,
        compiler_params=pltpu.CompilerParams(
            needs_layout_passes=False, use_tc_tiling_on_sc=False),
        scratch_types=[
            pltpu.VMEM((CHUNKS_PER_WORKER, CHUNK), jnp.int32),
            pltpu.VMEM((CHUNKS_PER_WORKER, CHUNK), jnp.int32),
            pltpu.VMEM((DIM, CHUNK), jnp.float32),
            pltpu.VMEM((DIM, CHUNK), jnp.float32),
            pltpu.VMEM((ROWS_PER_WORKER,), jnp.float32),
            pltpu.VMEM((ROWS_PER_WORKER,), jnp.float32),
            pltpu.VMEM((16,), jnp.float32),
            pltpu.VMEM((ROWS_PER_WORKER,), jnp.float32),
            pltpu.SemaphoreType.DMA,
        ],
    )
    return run(ui2, ii2, ut_t, it_t, user_bias, item_bias, gb16)


def kernel(user_indices, item_indices, user_table, item_table, user_bias,
           item_bias, global_bias):
    ui2 = user_indices.astype(jnp.int32).reshape(NCHUNKS, CHUNK)
    ii2 = item_indices.astype(jnp.int32).reshape(NCHUNKS, CHUNK)
    ut_t = user_table.T
    it_t = item_table.T
    gb16 = jnp.broadcast_to(global_bias.astype(jnp.float32), (16,))
    return _mfbias_call(ui2, ii2, ut_t, it_t, user_bias, item_bias, gb16)
